# baseline (speedup 1.0000x reference)
import functools

import jax
import jax.numpy as jnp
from jax import lax
from jax.experimental import pallas as pl
from jax.experimental.pallas import tpu as pltpu
from jax.experimental.pallas import tpu_sc as plsc

_LANES = 16
_SUBLANES = 8


def _sc_geometry():
    try:
        info = plsc.get_sparse_core_info()
        return info.num_cores, info.num_subcores
    except Exception:
        return 2, 16


def _make_kernel(rows, cols, num_types):
    nc, ns = _sc_geometry()
    nw = nc * ns
    assert rows % _SUBLANES == 0 and cols % (nw * 128) == 0
    lanes_per_w = cols // nw
    chunk_rows = None
    for cand in (40, 24, 16, 8):
        if rows % cand == 0 and 6 * cand * lanes_per_w * 4 <= 490_000:
            chunk_rows = cand
            break
    assert chunk_rows is not None and chunk_rows % _SUBLANES == 0
    n_chunks = rows // chunk_rows
    assert n_chunks >= 3

    mesh = plsc.VectorSubcoreMesh(core_axis_name="c", subcore_axis_name="s")

    @functools.partial(
        pl.kernel,
        mesh=mesh,
        out_type=jax.ShapeDtypeStruct((rows, cols), jnp.float32),
        compiler_params=pltpu.CompilerParams(
            needs_layout_passes=False, use_tc_tiling_on_sc=True),
        scratch_types=[
            pltpu.VMEM((max(num_types, 128),), jnp.float32),
            pltpu.VMEM((chunk_rows, lanes_per_w), jnp.int32),
            pltpu.VMEM((chunk_rows, lanes_per_w), jnp.int32),
            pltpu.VMEM((chunk_rows, lanes_per_w), jnp.float32),
            pltpu.VMEM((chunk_rows, lanes_per_w), jnp.float32),
            pltpu.VMEM((chunk_rows, lanes_per_w), jnp.float32),
            pltpu.VMEM((chunk_rows, lanes_per_w), jnp.float32),
            pltpu.SemaphoreType.DMA,
            pltpu.SemaphoreType.DMA,
            pltpu.SemaphoreType.DMA,
            pltpu.SemaphoreType.DMA,
            pltpu.SemaphoreType.DMA,
            pltpu.SemaphoreType.DMA,
        ],
    )
    def run(num_hbm, eng_hbm, tab_hbm, out_hbm, tab_v,
            num0, num1, eng0, eng1, out0, out1,
            ns0, ns1, es0, es1, ss0, ss1):
        nums, engs, outs = (num0, num1), (eng0, eng1), (out0, out1)
        nsem, esem, ssem = (ns0, ns1), (es0, es1), (ss0, ss1)
        wid = lax.axis_index("s") * nc + lax.axis_index("c")
        lane0 = wid * lanes_per_w
        pltpu.sync_copy(tab_hbm, tab_v.at[pl.ds(0, num_types)])
        t0 = tab_v[pl.ds(0, _LANES)]
        t1 = tab_v[pl.ds(_LANES, _LANES)]
        t2 = tab_v[pl.ds(2 * _LANES, _LANES)]
        t3 = tab_v[pl.ds(3 * _LANES, _LANES)]

        _dnums = lax.GatherDimensionNumbers(
            offset_dims=(), collapsed_slice_dims=(0,), start_index_map=(0,))

        def _vgather(tab, lo):
            return lax.gather(
                tab, lo[:, None], dimension_numbers=_dnums, slice_sizes=(1,),
                mode=lax.GatherScatterMode.PROMISE_IN_BOUNDS)

        def lookup(idx):
            lo = lax.bitwise_and(idx, _LANES - 1)
            hi = lax.shift_right_logical(idx, 4)
            g0 = _vgather(t0, lo)
            g1 = _vgather(t1, lo)
            g2 = _vgather(t2, lo)
            g3 = _vgather(t3, lo)
            v01 = jnp.where(hi == 0, g0, g1)
            v23 = jnp.where(hi == 2, g2, g3)
            return jnp.where(hi < 2, v01, v23)

        def hslice(c):
            return (pl.ds(c * chunk_rows, chunk_rows), pl.ds(lane0, lanes_per_w))

        def issue_loads(c, b):
            s = hslice(c)
            pltpu.async_copy(num_hbm.at[s], nums[b], nsem[b])
            pltpu.async_copy(eng_hbm.at[s], engs[b], esem[b])

        def wait_loads(c, b):
            s = hslice(c)
            pltpu.make_async_copy(num_hbm.at[s], nums[b], nsem[b]).wait()
            pltpu.make_async_copy(eng_hbm.at[s], engs[b], esem[b]).wait()

        n_sub = chunk_rows // _SUBLANES

        def wait_store(c, b):
            for sub in range(n_sub):
                sub_s = (pl.ds(c * chunk_rows + sub * _SUBLANES, _SUBLANES),
                         pl.ds(lane0, lanes_per_w))
                pltpu.make_async_copy(
                    outs[b].at[pl.ds(sub * _SUBLANES, _SUBLANES)],
                    out_hbm.at[sub_s], ssem[b]).wait()

        def compute_store(c, b):
            nv, ev, ov = nums[b], engs[b], outs[b]
            for sub in range(n_sub):
                @plsc.parallel_loop(0, lanes_per_w, step=_LANES, unroll=2)
                def body(cs):
                    for r in range(sub * _SUBLANES, (sub + 1) * _SUBLANES):
                        s = (r, pl.ds(cs, _LANES))
                        ov[s] = ev[s] + lookup(nv[s])
                sub_s = (pl.ds(c * chunk_rows + sub * _SUBLANES, _SUBLANES),
                         pl.ds(lane0, lanes_per_w))
                pltpu.async_copy(
                    ov.at[pl.ds(sub * _SUBLANES, _SUBLANES)],
                    out_hbm.at[sub_s], ssem[b])

        issue_loads(0, 0)

        def pair(g, carry):
            for b in range(2):
                c = 2 * g + b

                @pl.when(c + 1 < n_chunks)
                def _():
                    issue_loads(c + 1, 1 - b)

                wait_loads(c, b)

                @pl.when(c >= 2)
                def _():
                    wait_store(c - 2, b)

                compute_store(c, b)
            return carry

        lax.fori_loop(0, n_chunks // 2, pair, 0)
        if n_chunks % 2:
            c = n_chunks - 1
            wait_loads(c, 0)
            wait_store(c - 2, 0)
            compute_store(c, 0)
        wait_store(n_chunks - 1, (n_chunks - 1) % 2)
        wait_store(n_chunks - 2, (n_chunks - 2) % 2)

    return run


def kernel(numbers, energy, shifts_weight):
    b, l = energy.shape
    tab_flat = shifts_weight.reshape(-1)
    out_t = _make_kernel(l, b, tab_flat.shape[0])(
        numbers.astype(jnp.int32).T, energy.T, tab_flat)
    return out_t.T


# speedup vs baseline: 1.0343x; 1.0343x over previous
import functools

import jax
import jax.numpy as jnp
from jax import lax
from jax.experimental import pallas as pl
from jax.experimental.pallas import tpu as pltpu
from jax.experimental.pallas import tpu_sc as plsc

_LANES = 16
_SUBLANES = 8


def _sc_geometry():
    try:
        info = plsc.get_sparse_core_info()
        return info.num_cores, info.num_subcores
    except Exception:
        return 2, 16


def _make_kernel(rows, cols, num_types):
    nc, ns = _sc_geometry()
    nw = nc * ns
    assert rows % _SUBLANES == 0 and cols % (nw * 128) == 0
    lanes_per_w = cols // nw
    chunk_rows = None
    for cand in (40, 24, 16, 8):
        if rows % cand == 0 and 6 * cand * lanes_per_w * 4 <= 490_000:
            chunk_rows = cand
            break
    assert chunk_rows is not None and chunk_rows % _SUBLANES == 0
    n_chunks = rows // chunk_rows
    assert n_chunks >= 3

    mesh = plsc.VectorSubcoreMesh(core_axis_name="c", subcore_axis_name="s")

    @functools.partial(
        pl.kernel,
        mesh=mesh,
        out_type=jax.ShapeDtypeStruct((rows, cols), jnp.float32),
        compiler_params=pltpu.CompilerParams(
            needs_layout_passes=False, use_tc_tiling_on_sc=True),
        scratch_types=[
            pltpu.VMEM((max(num_types, 128),), jnp.float32),
            pltpu.VMEM((chunk_rows, lanes_per_w), jnp.int32),
            pltpu.VMEM((chunk_rows, lanes_per_w), jnp.int32),
            pltpu.VMEM((chunk_rows, lanes_per_w), jnp.float32),
            pltpu.VMEM((chunk_rows, lanes_per_w), jnp.float32),
            pltpu.VMEM((chunk_rows, lanes_per_w), jnp.float32),
            pltpu.VMEM((chunk_rows, lanes_per_w), jnp.float32),
            pltpu.SemaphoreType.DMA,
            pltpu.SemaphoreType.DMA,
            pltpu.SemaphoreType.DMA,
            pltpu.SemaphoreType.DMA,
            pltpu.SemaphoreType.DMA,
            pltpu.SemaphoreType.DMA,
        ],
    )
    def run(num_hbm, eng_hbm, tab_hbm, out_hbm, tab_v,
            num0, num1, eng0, eng1, out0, out1,
            ns0, ns1, es0, es1, ss0, ss1):
        nums, engs, outs = (num0, num1), (eng0, eng1), (out0, out1)
        nsem, esem, ssem = (ns0, ns1), (es0, es1), (ss0, ss1)
        wid = lax.axis_index("s") * nc + lax.axis_index("c")
        lane0 = wid * lanes_per_w
        pltpu.sync_copy(tab_hbm, tab_v.at[pl.ds(0, num_types)])

        def hslice(c):
            return (pl.ds(c * chunk_rows, chunk_rows), pl.ds(lane0, lanes_per_w))

        def issue_loads(c, b):
            s = hslice(c)
            pltpu.async_copy(num_hbm.at[s], nums[b], nsem[b])
            pltpu.async_copy(eng_hbm.at[s], engs[b], esem[b])

        def wait_loads(c, b):
            s = hslice(c)
            pltpu.make_async_copy(num_hbm.at[s], nums[b], nsem[b]).wait()
            pltpu.make_async_copy(eng_hbm.at[s], engs[b], esem[b]).wait()

        n_sub = chunk_rows // _SUBLANES

        def wait_store(c, b):
            for sub in range(n_sub):
                sub_s = (pl.ds(c * chunk_rows + sub * _SUBLANES, _SUBLANES),
                         pl.ds(lane0, lanes_per_w))
                pltpu.make_async_copy(
                    outs[b].at[pl.ds(sub * _SUBLANES, _SUBLANES)],
                    out_hbm.at[sub_s], ssem[b]).wait()

        def compute_store(c, b):
            nv, ev, ov = nums[b], engs[b], outs[b]
            for sub in range(n_sub):
                @plsc.parallel_loop(0, lanes_per_w, step=_LANES, unroll=2)
                def body(cs):
                    for r in range(sub * _SUBLANES, (sub + 1) * _SUBLANES):
                        s = (r, pl.ds(cs, _LANES))
                        vals = plsc.load_gather(tab_v, [nv[s]])
                        ov[s] = ev[s] + vals
                sub_s = (pl.ds(c * chunk_rows + sub * _SUBLANES, _SUBLANES),
                         pl.ds(lane0, lanes_per_w))
                pltpu.async_copy(
                    ov.at[pl.ds(sub * _SUBLANES, _SUBLANES)],
                    out_hbm.at[sub_s], ssem[b])

        issue_loads(0, 0)

        def pair(g, carry):
            for b in range(2):
                c = 2 * g + b

                @pl.when(c + 1 < n_chunks)
                def _():
                    issue_loads(c + 1, 1 - b)

                wait_loads(c, b)

                @pl.when(c >= 2)
                def _():
                    wait_store(c - 2, b)

                compute_store(c, b)
            return carry

        lax.fori_loop(0, n_chunks // 2, pair, 0)
        if n_chunks % 2:
            c = n_chunks - 1
            wait_loads(c, 0)
            wait_store(c - 2, 0)
            compute_store(c, 0)
        wait_store(n_chunks - 1, (n_chunks - 1) % 2)
        wait_store(n_chunks - 2, (n_chunks - 2) % 2)

    return run


def kernel(numbers, energy, shifts_weight):
    b, l = energy.shape
    tab_flat = shifts_weight.reshape(-1)
    out_t = _make_kernel(l, b, tab_flat.shape[0])(
        numbers.astype(jnp.int32).T, energy.T, tab_flat)
    return out_t.T


# speedup vs baseline: 1.0382x; 1.0038x over previous
import functools

import jax
import jax.numpy as jnp
from jax import lax
from jax.experimental import pallas as pl
from jax.experimental.pallas import tpu as pltpu
from jax.experimental.pallas import tpu_sc as plsc

_LANES = 16
_SUBLANES = 8


def _sc_geometry():
    try:
        info = plsc.get_sparse_core_info()
        return info.num_cores, info.num_subcores
    except Exception:
        return 2, 16


def _make_kernel(rows, cols, num_types):
    nc, ns = _sc_geometry()
    nw = nc * ns
    assert rows % _SUBLANES == 0 and cols % (nw * 128) == 0
    lanes_per_w = cols // nw
    chunk_rows = None
    for cand in (40, 24, 16, 8):
        if rows % cand == 0 and 6 * cand * lanes_per_w * 4 <= 490_000:
            chunk_rows = cand
            break
    assert chunk_rows is not None and chunk_rows % _SUBLANES == 0
    n_chunks = rows // chunk_rows
    assert n_chunks >= 3

    mesh = plsc.VectorSubcoreMesh(core_axis_name="c", subcore_axis_name="s")

    @functools.partial(
        pl.kernel,
        mesh=mesh,
        out_type=jax.ShapeDtypeStruct((rows, cols), jnp.float32),
        compiler_params=pltpu.CompilerParams(
            needs_layout_passes=False, use_tc_tiling_on_sc=True),
        scratch_types=[
            pltpu.VMEM((max(num_types, 128),), jnp.float32),
            pltpu.VMEM((chunk_rows, lanes_per_w), jnp.int32),
            pltpu.VMEM((chunk_rows, lanes_per_w), jnp.int32),
            pltpu.VMEM((chunk_rows, lanes_per_w), jnp.float32),
            pltpu.VMEM((chunk_rows, lanes_per_w), jnp.float32),
            pltpu.VMEM((chunk_rows, lanes_per_w), jnp.float32),
            pltpu.VMEM((chunk_rows, lanes_per_w), jnp.float32),
            pltpu.SemaphoreType.DMA,
            pltpu.SemaphoreType.DMA,
            pltpu.SemaphoreType.DMA,
            pltpu.SemaphoreType.DMA,
            pltpu.SemaphoreType.DMA,
            pltpu.SemaphoreType.DMA,
        ],
    )
    def run(num_hbm, eng_hbm, tab_hbm, out_hbm, tab_v,
            num0, num1, eng0, eng1, out0, out1,
            ns0, ns1, es0, es1, ss0, ss1):
        nums, engs, outs = (num0, num1), (eng0, eng1), (out0, out1)
        nsem, esem, ssem = (ns0, ns1), (es0, es1), (ss0, ss1)
        wid = lax.axis_index("s") * nc + lax.axis_index("c")
        lane0 = wid * lanes_per_w
        pltpu.sync_copy(tab_hbm, tab_v.at[pl.ds(0, num_types)])

        def hslice(c):
            return (pl.ds(c * chunk_rows, chunk_rows), pl.ds(lane0, lanes_per_w))

        def issue_loads(c, b):
            s = hslice(c)
            pltpu.async_copy(num_hbm.at[s], nums[b], nsem[b])
            pltpu.async_copy(eng_hbm.at[s], engs[b], esem[b])

        def wait_loads(c, b):
            s = hslice(c)
            pltpu.make_async_copy(num_hbm.at[s], nums[b], nsem[b]).wait()
            pltpu.make_async_copy(eng_hbm.at[s], engs[b], esem[b]).wait()

        n_sub = chunk_rows // _SUBLANES

        def wait_store(c, b):
            for sub in range(n_sub):
                sub_s = (pl.ds(c * chunk_rows + sub * _SUBLANES, _SUBLANES),
                         pl.ds(lane0, lanes_per_w))
                pltpu.make_async_copy(
                    outs[b].at[pl.ds(sub * _SUBLANES, _SUBLANES)],
                    out_hbm.at[sub_s], ssem[b]).wait()

        def compute_store(c, b):
            nv, ev, ov = nums[b], engs[b], outs[b]
            for sub in range(n_sub):
                @plsc.parallel_loop(0, lanes_per_w, step=_LANES, unroll=4)
                def body(cs):
                    for r in range(sub * _SUBLANES, (sub + 1) * _SUBLANES):
                        s = (r, pl.ds(cs, _LANES))
                        vals = plsc.load_gather(tab_v, [nv[s]])
                        ov[s] = ev[s] + vals
                sub_s = (pl.ds(c * chunk_rows + sub * _SUBLANES, _SUBLANES),
                         pl.ds(lane0, lanes_per_w))
                pltpu.async_copy(
                    ov.at[pl.ds(sub * _SUBLANES, _SUBLANES)],
                    out_hbm.at[sub_s], ssem[b])

        issue_loads(0, 0)

        def pair(g, carry):
            for b in range(2):
                c = 2 * g + b

                @pl.when(c + 1 < n_chunks)
                def _():
                    issue_loads(c + 1, 1 - b)

                wait_loads(c, b)

                @pl.when(c >= 2)
                def _():
                    wait_store(c - 2, b)

                compute_store(c, b)
            return carry

        lax.fori_loop(0, n_chunks // 2, pair, 0)
        if n_chunks % 2:
            c = n_chunks - 1
            wait_loads(c, 0)
            wait_store(c - 2, 0)
            compute_store(c, 0)
        wait_store(n_chunks - 1, (n_chunks - 1) % 2)
        wait_store(n_chunks - 2, (n_chunks - 2) % 2)

    return run


def kernel(numbers, energy, shifts_weight):
    b, l = energy.shape
    tab_flat = shifts_weight.reshape(-1)
    out_t = _make_kernel(l, b, tab_flat.shape[0])(
        numbers.astype(jnp.int32).T, energy.T, tab_flat)
    return out_t.T
